# Initial kernel scaffold; baseline (speedup 1.0000x reference)
import jax
import jax.numpy as jnp
from jax.experimental import pallas as pl


def kernel(x, edge_index, edge_weight, W1l, b1l, W1r, gamma, beta, W2l, b2l, W2r):
    raise NotImplementedError("write your pallas kernel here")


# speedup vs baseline: 5.3368x; 5.3368x over previous
import functools

import jax
import jax.numpy as jnp
from jax import lax
from jax.experimental import pallas as pl
from jax.experimental.pallas import tpu as pltpu
from jax.experimental.pallas import tpu_sc as plsc

N = 10000
E = 320000
D = 128
EPS = 1e-5

NC = 2
NS = 16
NW = NC * NS
EPW = E // NW
C = 80
NCHUNK = EPW // C
CPS = 25
NSLAB = NCHUNK // CPS
NP = 10240
RPT = NP // NS
LG = D // 16


def _segsum_body(*refs):
    (table, src3, dst3, wgt3, zrows, out,
     src_v, dst_v, w_v, rows_v, sem, acc_sh) = refs

    c = lax.axis_index("c")
    s = lax.axis_index("s")
    wid = s * NC + c

    pltpu.sync_copy(zrows, acc_sh.at[pl.ds(s * RPT, RPT)])
    plsc.subcore_barrier()

    def slab_body(sl, carry):
        pltpu.sync_copy(src3.at[wid, sl], src_v)
        pltpu.sync_copy(dst3.at[wid, sl], dst_v)
        pltpu.sync_copy(wgt3.at[wid, sl], w_v)

        def chunk_body(k, carry1):
            pltpu.async_copy(table.at[src_v.at[k]], rows_v, sem).wait()

            def grp_body(g, carry2):
                w_reg = w_v[pl.ds(k * C + g * 16, 16)]
                for r in range(16):
                    wv = lax.gather(
                        w_reg, jnp.full((16, 1), r, jnp.int32),
                        lax.GatherDimensionNumbers(offset_dims=(),
                                                   collapsed_slice_dims=(0,),
                                                   start_index_map=(0,)),
                        slice_sizes=(1,),
                        mode=lax.GatherScatterMode.PROMISE_IN_BOUNDS)
                    row = g * 16 + r
                    for j in range(LG):
                        rows_v[row, pl.ds(j * 16, 16)] = (
                            rows_v[row, pl.ds(j * 16, 16)] * wv)
                return carry2

            lax.fori_loop(0, C // 16, grp_body, 0)

            pltpu.sync_copy(rows_v, acc_sh.at[dst_v.at[k]], add=True)
            return carry1

        lax.fori_loop(0, CPS, chunk_body, 0)
        return carry

    lax.fori_loop(0, NSLAB, slab_body, 0)
    plsc.subcore_barrier()

    pltpu.sync_copy(acc_sh.at[pl.ds(s * RPT, RPT)],
                    out.at[c, pl.ds(s * RPT, RPT)])


def _count_body(*refs):
    (dst3, zrows, out, dst_v, ones_v, acc_sh) = refs

    c = lax.axis_index("c")
    s = lax.axis_index("s")
    wid = s * NC + c

    pltpu.sync_copy(zrows, acc_sh.at[pl.ds(s * RPT, RPT)])

    def ones_row(r, carry):
        for j in range(LG):
            ones_v[r, pl.ds(j * 16, 16)] = jnp.full((16,), 1.0, jnp.float32)
        return carry

    lax.fori_loop(0, C, ones_row, 0)
    plsc.subcore_barrier()

    def slab_body(sl, carry):
        pltpu.sync_copy(dst3.at[wid, sl], dst_v)

        def chunk_body(k, carry1):
            pltpu.sync_copy(ones_v, acc_sh.at[dst_v.at[k]], add=True)
            return carry1

        lax.fori_loop(0, CPS, chunk_body, 0)
        return carry

    lax.fori_loop(0, NSLAB, slab_body, 0)
    plsc.subcore_barrier()
    pltpu.sync_copy(acc_sh.at[pl.ds(s * RPT, RPT)],
                    out.at[c, pl.ds(s * RPT, RPT)])


def _make_segsum():
    mesh = plsc.VectorSubcoreMesh(core_axis_name="c", subcore_axis_name="s",
                                  num_cores=NC, num_subcores=NS)
    return pl.kernel(
        _segsum_body,
        out_type=jax.ShapeDtypeStruct((NC, NP, D), jnp.float32),
        mesh=mesh,
        scratch_types=[
            pltpu.VMEM((CPS, C), jnp.int32),
            pltpu.VMEM((CPS, C), jnp.int32),
            pltpu.VMEM((CPS * C,), jnp.float32),
            pltpu.VMEM((C, D), jnp.float32),
            pltpu.SemaphoreType.DMA,
            pltpu.VMEM_SHARED((NP, D), jnp.float32),
        ],
        name="sc_segsum",
    )


def _make_count():
    mesh = plsc.VectorSubcoreMesh(core_axis_name="c", subcore_axis_name="s",
                                  num_cores=NC, num_subcores=NS)
    return pl.kernel(
        _count_body,
        out_type=jax.ShapeDtypeStruct((NC, NP, D), jnp.float32),
        mesh=mesh,
        scratch_types=[
            pltpu.VMEM((CPS, C), jnp.int32),
            pltpu.VMEM((C, D), jnp.float32),
            pltpu.VMEM_SHARED((NP, D), jnp.float32),
        ],
        name="sc_count",
    )


_make_segsum = functools.lru_cache(maxsize=None)(_make_segsum)
_make_count = functools.lru_cache(maxsize=None)(_make_count)


R = 1000
GRID = N // R


def _dense1_body(sum0, sum1, cnt0, cnt1, x, wl_t, bl, wr_t, hpre, ssum, ssq):
    i = pl.program_id(0)
    cnt = jnp.maximum(cnt0[:, 0:1] + cnt1[:, 0:1], 1.0)
    aggr = (sum0[...] + sum1[...]) / cnt
    hp = (jnp.dot(aggr, wl_t[...], preferred_element_type=jnp.float32)
          + bl[...]
          + jnp.dot(x[...], wr_t[...], preferred_element_type=jnp.float32))
    hpre[...] = hp

    @pl.when(i == 0)
    def _():
        ssum[...] = jnp.zeros_like(ssum)
        ssq[...] = jnp.zeros_like(ssq)

    ssum[...] += jnp.sum(hp, axis=0, keepdims=True)
    ssq[...] += jnp.sum(hp * hp, axis=0, keepdims=True)


def _bn_relu_body(hpre, ssum, ssq, gamma, beta, h):
    mu = ssum[...] / N
    var = ssq[...] / N - mu * mu
    inv = lax.rsqrt(var + EPS) * gamma[...]
    h[...] = jnp.maximum((hpre[...] - mu) * inv + beta[...], 0.0)


def _dense2_body(sum0, sum1, cnt0, cnt1, h, wl_t, bl, wr_t, out):
    cnt = jnp.maximum(cnt0[:, 0:1] + cnt1[:, 0:1], 1.0)
    aggr = (sum0[...] + sum1[...]) / cnt
    out[...] = (jnp.dot(aggr, wl_t[...], preferred_element_type=jnp.float32)
                + bl[...]
                + jnp.dot(h[...], wr_t[...], preferred_element_type=jnp.float32))


_row_blk = pl.BlockSpec((R, D), lambda i: (i, 0))
_sum_blk = pl.BlockSpec((R, D), lambda i: (i, 0))
_full_w = pl.BlockSpec((D, D), lambda i: (0, 0))
_full_v = pl.BlockSpec((1, D), lambda i: (0, 0))

_dense1 = pl.pallas_call(
    _dense1_body,
    grid=(GRID,),
    in_specs=[_sum_blk, _sum_blk, _sum_blk, _sum_blk, _row_blk,
              _full_w, _full_v, _full_w],
    out_specs=[_row_blk, _full_v, _full_v],
    out_shape=[jax.ShapeDtypeStruct((N, D), jnp.float32),
               jax.ShapeDtypeStruct((1, D), jnp.float32),
               jax.ShapeDtypeStruct((1, D), jnp.float32)],
)

_bn_relu = pl.pallas_call(
    _bn_relu_body,
    grid=(GRID,),
    in_specs=[_row_blk, _full_v, _full_v, _full_v, _full_v],
    out_specs=_row_blk,
    out_shape=jax.ShapeDtypeStruct((N, D), jnp.float32),
)

_dense2 = pl.pallas_call(
    _dense2_body,
    grid=(GRID,),
    in_specs=[_sum_blk, _sum_blk, _sum_blk, _sum_blk, _row_blk,
              _full_w, _full_v, _full_w],
    out_specs=_row_blk,
    out_shape=jax.ShapeDtypeStruct((N, D), jnp.float32),
)


def kernel(x, edge_index, edge_weight, W1l, b1l, W1r, gamma, beta, W2l, b2l, W2r):
    src = edge_index[0].astype(jnp.int32).reshape(NW, NSLAB, CPS, C)
    dst = edge_index[1].astype(jnp.int32).reshape(NW, NSLAB, CPS, C)
    wgt = edge_weight.astype(jnp.float32).reshape(NW, NSLAB, CPS * C)
    zrows = jnp.zeros((RPT, D), jnp.float32)

    segsum = _make_segsum()
    cnt = _make_count()(dst, zrows)
    sum1 = segsum(x, src, dst, wgt, zrows)
    hpre, ssum, ssq = _dense1(sum1[0], sum1[1], cnt[0], cnt[1], x,
                              W1l.T, b1l.reshape(1, D), W1r.T)
    h = _bn_relu(hpre, ssum, ssq, gamma.reshape(1, D), beta.reshape(1, D))
    sum2 = segsum(h, src, dst, wgt, zrows)
    out = _dense2(sum2[0], sum2[1], cnt[0], cnt[1], h,
                  W2l.T, b2l.reshape(1, D), W2r.T)
    return out


# speedup vs baseline: 7.7306x; 1.4485x over previous
import functools

import jax
import jax.numpy as jnp
from jax import lax
from jax.experimental import pallas as pl
from jax.experimental.pallas import tpu as pltpu
from jax.experimental.pallas import tpu_sc as plsc

N = 10000
E = 320000
D = 128
EPS = 1e-5

NC = 2
NS = 16
NW = NC * NS
EPW = E // NW
C = 80
NCHUNK = EPW // C
CPS = 25
NSLAB = NCHUNK // CPS
NP = 10240
RPT = NP // NS
LG = D // 16


def _mult_rows(rows_ref, w_v, k):
    def grp_body(g, carry):
        w_reg = w_v[pl.ds(k * C + g * 16, 16)]
        for r in range(16):
            wv = lax.gather(
                w_reg, jnp.full((16, 1), r, jnp.int32),
                lax.GatherDimensionNumbers(offset_dims=(),
                                           collapsed_slice_dims=(0,),
                                           start_index_map=(0,)),
                slice_sizes=(1,),
                mode=lax.GatherScatterMode.PROMISE_IN_BOUNDS)
            row = g * 16 + r
            for j in range(LG):
                rows_ref[row, pl.ds(j * 16, 16)] = (
                    rows_ref[row, pl.ds(j * 16, 16)] * wv)
        return carry

    lax.fori_loop(0, C // 16, grp_body, 0)


def _segsum_body(*refs):
    (table, src3, dst3, wgt3, zrows, out,
     src_v, dst_v, w_v, rows_a, rows_b, sga, sgb, acc_sh) = refs

    c = lax.axis_index("c")
    s = lax.axis_index("s")
    wid = s * NC + c

    pltpu.sync_copy(zrows, acc_sh.at[pl.ds(s * RPT, RPT)])
    plsc.subcore_barrier()

    def slab_body(sl, carry):
        pltpu.sync_copy(src3.at[wid, sl], src_v)
        pltpu.sync_copy(dst3.at[wid, sl], dst_v)
        pltpu.sync_copy(wgt3.at[wid, sl], w_v)

        pltpu.async_copy(table.at[src_v.at[0]], rows_a, sga)

        def pair_body(p, carry1):
            ka = 2 * p
            kb = 2 * p + 1
            pltpu.async_copy(table.at[src_v.at[kb]], rows_b, sgb)
            pltpu.make_async_copy(table.at[src_v.at[ka]], rows_a, sga).wait()
            _mult_rows(rows_a, w_v, ka)
            pltpu.sync_copy(rows_a, acc_sh.at[dst_v.at[ka]], add=True)
            pltpu.async_copy(table.at[src_v.at[ka + 2]], rows_a, sga)
            pltpu.make_async_copy(table.at[src_v.at[kb]], rows_b, sgb).wait()
            _mult_rows(rows_b, w_v, kb)
            pltpu.sync_copy(rows_b, acc_sh.at[dst_v.at[kb]], add=True)
            return carry1

        lax.fori_loop(0, CPS // 2, pair_body, 0)

        kt = CPS - 1
        pltpu.make_async_copy(table.at[src_v.at[kt]], rows_a, sga).wait()
        _mult_rows(rows_a, w_v, kt)
        pltpu.sync_copy(rows_a, acc_sh.at[dst_v.at[kt]], add=True)
        return carry

    lax.fori_loop(0, NSLAB, slab_body, 0)
    plsc.subcore_barrier()

    pltpu.sync_copy(acc_sh.at[pl.ds(s * RPT, RPT)],
                    out.at[c, pl.ds(s * RPT, RPT)])


def _count_body(*refs):
    (dst3, zrows, out, dst_v, ones_v, acc_sh) = refs

    c = lax.axis_index("c")
    s = lax.axis_index("s")
    wid = s * NC + c

    pltpu.sync_copy(zrows, acc_sh.at[pl.ds(s * RPT, RPT)])

    def ones_row(r, carry):
        for j in range(LG):
            ones_v[r, pl.ds(j * 16, 16)] = jnp.full((16,), 1.0, jnp.float32)
        return carry

    lax.fori_loop(0, C, ones_row, 0)
    plsc.subcore_barrier()

    def slab_body(sl, carry):
        pltpu.sync_copy(dst3.at[wid, sl], dst_v)

        def chunk_body(k, carry1):
            pltpu.sync_copy(ones_v, acc_sh.at[dst_v.at[k]], add=True)
            return carry1

        lax.fori_loop(0, CPS, chunk_body, 0)
        return carry

    lax.fori_loop(0, NSLAB, slab_body, 0)
    plsc.subcore_barrier()
    pltpu.sync_copy(acc_sh.at[pl.ds(s * RPT, RPT)],
                    out.at[c, pl.ds(s * RPT, RPT)])


def _make_segsum():
    mesh = plsc.VectorSubcoreMesh(core_axis_name="c", subcore_axis_name="s",
                                  num_cores=NC, num_subcores=NS)
    return pl.kernel(
        _segsum_body,
        out_type=jax.ShapeDtypeStruct((NC, NP, D), jnp.float32),
        mesh=mesh,
        scratch_types=[
            pltpu.VMEM((CPS, C), jnp.int32),
            pltpu.VMEM((CPS, C), jnp.int32),
            pltpu.VMEM((CPS * C,), jnp.float32),
            pltpu.VMEM((C, D), jnp.float32),
            pltpu.VMEM((C, D), jnp.float32),
            pltpu.SemaphoreType.DMA,
            pltpu.SemaphoreType.DMA,
            pltpu.VMEM_SHARED((NP, D), jnp.float32),
        ],
        name="sc_segsum",
    )


def _make_count():
    mesh = plsc.VectorSubcoreMesh(core_axis_name="c", subcore_axis_name="s",
                                  num_cores=NC, num_subcores=NS)
    return pl.kernel(
        _count_body,
        out_type=jax.ShapeDtypeStruct((NC, NP, D), jnp.float32),
        mesh=mesh,
        scratch_types=[
            pltpu.VMEM((CPS, C), jnp.int32),
            pltpu.VMEM((C, D), jnp.float32),
            pltpu.VMEM_SHARED((NP, D), jnp.float32),
        ],
        name="sc_count",
    )


_make_segsum = functools.lru_cache(maxsize=None)(_make_segsum)
_make_count = functools.lru_cache(maxsize=None)(_make_count)


R = 1000
GRID = N // R


def _dense1_body(sum0, sum1, cnt0, cnt1, x, wl_t, bl, wr_t, hpre, ssum, ssq):
    i = pl.program_id(0)
    cnt = jnp.maximum(cnt0[:, 0:1] + cnt1[:, 0:1], 1.0)
    aggr = (sum0[...] + sum1[...]) / cnt
    hp = (jnp.dot(aggr, wl_t[...], preferred_element_type=jnp.float32)
          + bl[...]
          + jnp.dot(x[...], wr_t[...], preferred_element_type=jnp.float32))
    hpre[...] = hp

    @pl.when(i == 0)
    def _():
        ssum[...] = jnp.zeros_like(ssum)
        ssq[...] = jnp.zeros_like(ssq)

    ssum[...] += jnp.sum(hp, axis=0, keepdims=True)
    ssq[...] += jnp.sum(hp * hp, axis=0, keepdims=True)


def _bn_relu_body(hpre, ssum, ssq, gamma, beta, h):
    mu = ssum[...] / N
    var = ssq[...] / N - mu * mu
    inv = lax.rsqrt(var + EPS) * gamma[...]
    h[...] = jnp.maximum((hpre[...] - mu) * inv + beta[...], 0.0)


def _dense2_body(sum0, sum1, cnt0, cnt1, h, wl_t, bl, wr_t, out):
    cnt = jnp.maximum(cnt0[:, 0:1] + cnt1[:, 0:1], 1.0)
    aggr = (sum0[...] + sum1[...]) / cnt
    out[...] = (jnp.dot(aggr, wl_t[...], preferred_element_type=jnp.float32)
                + bl[...]
                + jnp.dot(h[...], wr_t[...], preferred_element_type=jnp.float32))


_row_blk = pl.BlockSpec((R, D), lambda i: (i, 0))
_sum_blk = pl.BlockSpec((R, D), lambda i: (i, 0))
_full_w = pl.BlockSpec((D, D), lambda i: (0, 0))
_full_v = pl.BlockSpec((1, D), lambda i: (0, 0))

_dense1 = pl.pallas_call(
    _dense1_body,
    grid=(GRID,),
    in_specs=[_sum_blk, _sum_blk, _sum_blk, _sum_blk, _row_blk,
              _full_w, _full_v, _full_w],
    out_specs=[_row_blk, _full_v, _full_v],
    out_shape=[jax.ShapeDtypeStruct((N, D), jnp.float32),
               jax.ShapeDtypeStruct((1, D), jnp.float32),
               jax.ShapeDtypeStruct((1, D), jnp.float32)],
)

_bn_relu = pl.pallas_call(
    _bn_relu_body,
    grid=(GRID,),
    in_specs=[_row_blk, _full_v, _full_v, _full_v, _full_v],
    out_specs=_row_blk,
    out_shape=jax.ShapeDtypeStruct((N, D), jnp.float32),
)

_dense2 = pl.pallas_call(
    _dense2_body,
    grid=(GRID,),
    in_specs=[_sum_blk, _sum_blk, _sum_blk, _sum_blk, _row_blk,
              _full_w, _full_v, _full_w],
    out_specs=_row_blk,
    out_shape=jax.ShapeDtypeStruct((N, D), jnp.float32),
)


def kernel(x, edge_index, edge_weight, W1l, b1l, W1r, gamma, beta, W2l, b2l, W2r):
    src = edge_index[0].astype(jnp.int32).reshape(NW, NSLAB, CPS, C)
    dst = edge_index[1].astype(jnp.int32).reshape(NW, NSLAB, CPS, C)
    wgt = edge_weight.astype(jnp.float32).reshape(NW, NSLAB, CPS * C)
    zrows = jnp.zeros((RPT, D), jnp.float32)

    segsum = _make_segsum()
    cnt = _make_count()(dst, zrows)
    sum1 = segsum(x, src, dst, wgt, zrows)
    hpre, ssum, ssq = _dense1(sum1[0], sum1[1], cnt[0], cnt[1], x,
                              W1l.T, b1l.reshape(1, D), W1r.T)
    h = _bn_relu(hpre, ssum, ssq, gamma.reshape(1, D), beta.reshape(1, D))
    sum2 = segsum(h, src, dst, wgt, zrows)
    out = _dense2(sum2[0], sum2[1], cnt[0], cnt[1], h,
                  W2l.T, b2l.reshape(1, D), W2r.T)
    return out


# speedup vs baseline: 7.7744x; 1.0057x over previous
import functools

import jax
import jax.numpy as jnp
from jax import lax
from jax.experimental import pallas as pl
from jax.experimental.pallas import tpu as pltpu
from jax.experimental.pallas import tpu_sc as plsc

N = 10000
E = 320000
D = 128
EPS = 1e-5

NC = 2
NS = 16
NW = NC * NS
EPW = E // NW
C = 80
NCHUNK = EPW // C
CPS = 25
NSLAB = NCHUNK // CPS
NP = 10240
RPT = NP // NS
LG = D // 16
WCNT = 4
NBUF = 3


def _mult_rows(rows_ref, w_v, k):
    def grp_body(g, carry):
        w_reg = w_v[pl.ds(k * C + g * 16, 16)]
        for r in range(16):
            wv = lax.gather(
                w_reg, jnp.full((16, 1), r, jnp.int32),
                lax.GatherDimensionNumbers(offset_dims=(),
                                           collapsed_slice_dims=(0,),
                                           start_index_map=(0,)),
                slice_sizes=(1,),
                mode=lax.GatherScatterMode.PROMISE_IN_BOUNDS)
            row = g * 16 + r
            for j in range(LG):
                rows_ref[row, pl.ds(j * 16, 16)] = (
                    rows_ref[row, pl.ds(j * 16, 16)] * wv)
        return carry

    lax.fori_loop(0, C // 16, grp_body, 0)


def _segsum_body(*refs):
    (table, src3, dst3, wgt3, zrows, out,
     src_v, dst_v, w_v, r0, r1, r2,
     sg0, sg1, sg2, ss0, ss1, ss2, acc_sh) = refs
    bufs = (r0, r1, r2)
    sg = (sg0, sg1, sg2)
    ss = (ss0, ss1, ss2)

    c = lax.axis_index("c")
    s = lax.axis_index("s")
    wid = s * NC + c

    pltpu.sync_copy(zrows, acc_sh.at[pl.ds(s * RPT, RPT)])
    plsc.subcore_barrier()

    def slab_body(sl, carry):
        pltpu.sync_copy(src3.at[wid, sl], src_v)
        pltpu.sync_copy(dst3.at[wid, sl], dst_v)
        pltpu.sync_copy(wgt3.at[wid, sl], w_v)

        for b in range(NBUF):
            pltpu.async_copy(table.at[src_v.at[b]], bufs[b], sg[b])

        def group_body(g, carry1):
            for b in range(NBUF):
                k = NBUF * g + b
                pltpu.make_async_copy(table.at[src_v.at[k]], bufs[b],
                                      sg[b]).wait()
                _mult_rows(bufs[b], w_v, k)
                pltpu.async_copy(bufs[b], acc_sh.at[dst_v.at[k]], ss[b],
                                 add=True)
            for b in range(NBUF):
                kn = NBUF * g + NBUF + b

                @pl.when(kn < CPS)
                def _():
                    pltpu.make_async_copy(bufs[b], acc_sh.at[dst_v.at[0]],
                                          ss[b]).wait()
                    pltpu.async_copy(table.at[src_v.at[kn]], bufs[b], sg[b])

            return carry1

        lax.fori_loop(0, CPS // NBUF, group_body, 0)

        kt = CPS - 1
        pltpu.make_async_copy(table.at[src_v.at[kt]], bufs[0], sg[0]).wait()
        _mult_rows(bufs[0], w_v, kt)
        pltpu.sync_copy(bufs[0], acc_sh.at[dst_v.at[kt]], add=True)
        for b in range(1, NBUF):
            pltpu.make_async_copy(bufs[b], acc_sh.at[dst_v.at[0]],
                                  ss[b]).wait()
        return carry

    lax.fori_loop(0, NSLAB, slab_body, 0)
    plsc.subcore_barrier()

    pltpu.sync_copy(acc_sh.at[pl.ds(s * RPT, RPT)],
                    out.at[c, pl.ds(s * RPT, RPT)])


def _count_body(*refs):
    (dst2, zrows, out, dst_v, ones_v, sem, acc_sh) = refs

    c = lax.axis_index("c")
    s = lax.axis_index("s")
    wid = s * NC + c

    pltpu.sync_copy(zrows, acc_sh.at[pl.ds(s * RPT, RPT)])
    pltpu.sync_copy(dst2.at[wid], dst_v)

    def ones_row(r, carry):
        for j in range(LG):
            ones_v[r, pl.ds(j * 16, 16)] = jnp.full((16,), 1.0, jnp.float32)
        return carry

    lax.fori_loop(0, C, ones_row, 0)
    plsc.subcore_barrier()

    def chunk_body(k, carry1):
        pltpu.async_copy(ones_v, acc_sh.at[dst_v.at[k]], sem, add=True)

        @pl.when(k >= WCNT)
        def _():
            pltpu.make_async_copy(ones_v, acc_sh.at[dst_v.at[0]], sem).wait()

        return carry1

    lax.fori_loop(0, NCHUNK, chunk_body, 0)
    for _ in range(WCNT):
        pltpu.make_async_copy(ones_v, acc_sh.at[dst_v.at[0]], sem).wait()
    plsc.subcore_barrier()
    pltpu.sync_copy(acc_sh.at[pl.ds(s * RPT, RPT)],
                    out.at[c, pl.ds(s * RPT, RPT)])


def _make_segsum():
    mesh = plsc.VectorSubcoreMesh(core_axis_name="c", subcore_axis_name="s",
                                  num_cores=NC, num_subcores=NS)
    return pl.kernel(
        _segsum_body,
        out_type=jax.ShapeDtypeStruct((NC, NP, D), jnp.float32),
        mesh=mesh,
        scratch_types=[
            pltpu.VMEM((CPS, C), jnp.int32),
            pltpu.VMEM((CPS, C), jnp.int32),
            pltpu.VMEM((CPS * C,), jnp.float32),
            pltpu.VMEM((C, D), jnp.float32),
            pltpu.VMEM((C, D), jnp.float32),
            pltpu.VMEM((C, D), jnp.float32),
            pltpu.SemaphoreType.DMA,
            pltpu.SemaphoreType.DMA,
            pltpu.SemaphoreType.DMA,
            pltpu.SemaphoreType.DMA,
            pltpu.SemaphoreType.DMA,
            pltpu.SemaphoreType.DMA,
            pltpu.VMEM_SHARED((NP, D), jnp.float32),
        ],
        name="sc_segsum",
    )


def _make_count():
    mesh = plsc.VectorSubcoreMesh(core_axis_name="c", subcore_axis_name="s",
                                  num_cores=NC, num_subcores=NS)
    return pl.kernel(
        _count_body,
        out_type=jax.ShapeDtypeStruct((NC, NP, D), jnp.float32),
        mesh=mesh,
        scratch_types=[
            pltpu.VMEM((NCHUNK, C), jnp.int32),
            pltpu.VMEM((C, D), jnp.float32),
            pltpu.SemaphoreType.DMA,
            pltpu.VMEM_SHARED((NP, D), jnp.float32),
        ],
        name="sc_count",
    )


_make_segsum = functools.lru_cache(maxsize=None)(_make_segsum)
_make_count = functools.lru_cache(maxsize=None)(_make_count)


R = 1000
GRID = N // R


def _dense1_body(sum0, sum1, cnt0, cnt1, x, wl_t, bl, wr_t, hpre, ssum, ssq):
    i = pl.program_id(0)
    cnt = jnp.maximum(cnt0[:, 0:1] + cnt1[:, 0:1], 1.0)
    aggr = (sum0[...] + sum1[...]) / cnt
    hp = (jnp.dot(aggr, wl_t[...], preferred_element_type=jnp.float32)
          + bl[...]
          + jnp.dot(x[...], wr_t[...], preferred_element_type=jnp.float32))
    hpre[...] = hp

    @pl.when(i == 0)
    def _():
        ssum[...] = jnp.zeros_like(ssum)
        ssq[...] = jnp.zeros_like(ssq)

    ssum[...] += jnp.sum(hp, axis=0, keepdims=True)
    ssq[...] += jnp.sum(hp * hp, axis=0, keepdims=True)


def _bn_relu_body(hpre, ssum, ssq, gamma, beta, h):
    mu = ssum[...] / N
    var = ssq[...] / N - mu * mu
    inv = lax.rsqrt(var + EPS) * gamma[...]
    h[...] = jnp.maximum((hpre[...] - mu) * inv + beta[...], 0.0)


def _dense2_body(sum0, sum1, cnt0, cnt1, h, wl_t, bl, wr_t, out):
    cnt = jnp.maximum(cnt0[:, 0:1] + cnt1[:, 0:1], 1.0)
    aggr = (sum0[...] + sum1[...]) / cnt
    out[...] = (jnp.dot(aggr, wl_t[...], preferred_element_type=jnp.float32)
                + bl[...]
                + jnp.dot(h[...], wr_t[...], preferred_element_type=jnp.float32))


_row_blk = pl.BlockSpec((R, D), lambda i: (i, 0))
_sum_blk = pl.BlockSpec((R, D), lambda i: (i, 0))
_full_w = pl.BlockSpec((D, D), lambda i: (0, 0))
_full_v = pl.BlockSpec((1, D), lambda i: (0, 0))

_dense1 = pl.pallas_call(
    _dense1_body,
    grid=(GRID,),
    in_specs=[_sum_blk, _sum_blk, _sum_blk, _sum_blk, _row_blk,
              _full_w, _full_v, _full_w],
    out_specs=[_row_blk, _full_v, _full_v],
    out_shape=[jax.ShapeDtypeStruct((N, D), jnp.float32),
               jax.ShapeDtypeStruct((1, D), jnp.float32),
               jax.ShapeDtypeStruct((1, D), jnp.float32)],
)

_bn_relu = pl.pallas_call(
    _bn_relu_body,
    grid=(GRID,),
    in_specs=[_row_blk, _full_v, _full_v, _full_v, _full_v],
    out_specs=_row_blk,
    out_shape=jax.ShapeDtypeStruct((N, D), jnp.float32),
)

_dense2 = pl.pallas_call(
    _dense2_body,
    grid=(GRID,),
    in_specs=[_sum_blk, _sum_blk, _sum_blk, _sum_blk, _row_blk,
              _full_w, _full_v, _full_w],
    out_specs=_row_blk,
    out_shape=jax.ShapeDtypeStruct((N, D), jnp.float32),
)


def kernel(x, edge_index, edge_weight, W1l, b1l, W1r, gamma, beta, W2l, b2l, W2r):
    src = edge_index[0].astype(jnp.int32).reshape(NW, NSLAB, CPS, C)
    dst = edge_index[1].astype(jnp.int32).reshape(NW, NSLAB, CPS, C)
    wgt = edge_weight.astype(jnp.float32).reshape(NW, NSLAB, CPS * C)
    zrows = jnp.zeros((RPT, D), jnp.float32)

    segsum = _make_segsum()
    dst2 = edge_index[1].astype(jnp.int32).reshape(NW, NCHUNK, C)
    cnt = _make_count()(dst2, zrows)
    sum1 = segsum(x, src, dst, wgt, zrows)
    hpre, ssum, ssq = _dense1(sum1[0], sum1[1], cnt[0], cnt[1], x,
                              W1l.T, b1l.reshape(1, D), W1r.T)
    h = _bn_relu(hpre, ssum, ssq, gamma.reshape(1, D), beta.reshape(1, D))
    sum2 = segsum(h, src, dst, wgt, zrows)
    out = _dense2(sum2[0], sum2[1], cnt[0], cnt[1], h,
                  W2l.T, b2l.reshape(1, D), W2r.T)
    return out
